# baseline (speedup 1.0000x reference)
import functools

import jax
import jax.numpy as jnp
from jax import lax
from jax.experimental import pallas as pl
from jax.experimental.pallas import tpu as pltpu
from jax.experimental.pallas import tpu_sc as plsc

BATCH = 16384
EMB = 8
NC = 2
NS = 16
NW = NC * NS
BPW = BATCH // NW
CHUNK = 128
NCHUNK = BPW // CHUNK
IDX_ROWS = BATCH // CHUNK


def _sc_gather(user2d, item2d, t_um, t_im, t_uf, t_if):
    mesh = plsc.VectorSubcoreMesh(core_axis_name="c", subcore_axis_name="s")
    out_t = [jax.ShapeDtypeStruct((BATCH, EMB), jnp.float32)] * 4

    @functools.partial(
        pl.kernel,
        mesh=mesh,
        out_type=out_t,
        compiler_params=pltpu.CompilerParams(use_tc_tiling_on_sc=False),
        scratch_types=[
            pltpu.VMEM((NCHUNK, CHUNK), jnp.int32),
            pltpu.VMEM((NCHUNK, CHUNK), jnp.int32),
            pltpu.VMEM((BPW, EMB), jnp.float32),
            pltpu.VMEM((BPW, EMB), jnp.float32),
            pltpu.VMEM((BPW, EMB), jnp.float32),
            pltpu.VMEM((BPW, EMB), jnp.float32),
            pltpu.SemaphoreType.DMA,
        ],
    )
    def k(u_hbm, i_hbm, um_hbm, im_hbm, uf_hbm, if_hbm,
          o_um, o_im, o_uf, o_if,
          uidx, iidx, r_um, r_im, r_uf, r_if, sem):
        wid = lax.axis_index("s") * NC + lax.axis_index("c")
        base = wid * BPW
        row0 = wid * NCHUNK
        pltpu.sync_copy(u_hbm.at[pl.ds(row0, NCHUNK)], uidx)
        pltpu.sync_copy(i_hbm.at[pl.ds(row0, NCHUNK)], iidx)
        copies = []
        for j in range(NCHUNK):
            sl = pl.ds(j * CHUNK, CHUNK)
            copies.append(pltpu.async_copy(um_hbm.at[uidx.at[j]], r_um.at[sl], sem))
            copies.append(pltpu.async_copy(im_hbm.at[iidx.at[j]], r_im.at[sl], sem))
            copies.append(pltpu.async_copy(uf_hbm.at[uidx.at[j]], r_uf.at[sl], sem))
            copies.append(pltpu.async_copy(if_hbm.at[iidx.at[j]], r_if.at[sl], sem))
        for c in copies:
            c.wait()
        osl = pl.ds(base, BPW)
        pltpu.sync_copy(r_um, o_um.at[osl])
        pltpu.sync_copy(r_im, o_im.at[osl])
        pltpu.sync_copy(r_uf, o_uf.at[osl])
        pltpu.sync_copy(r_if, o_if.at[osl])

    return k(user2d, item2d, t_um, t_im, t_uf, t_if)


BT = 2048


def _tc_body(um, im, uf, itf, w1u, w1i, b1r, w2, b2r, wah, waf, bar, out):
    h = jnp.maximum(
        jnp.dot(um[...], w1u[...], preferred_element_type=jnp.float32)
        + jnp.dot(im[...], w1i[...], preferred_element_type=jnp.float32)
        + b1r[...], 0.0)
    h2 = jnp.maximum(
        jnp.dot(h, w2[...], preferred_element_type=jnp.float32) + b2r[...], 0.0)
    mf = uf[...] * itf[...]
    logits = (jnp.dot(h2, wah[...], preferred_element_type=jnp.float32)
              + jnp.dot(mf, waf[...], preferred_element_type=jnp.float32)
              + bar[...])
    out[...] = jax.nn.sigmoid(logits)


def _tc_dense(u_mlp, i_mlp, u_mf, i_mf, w1u, w1i, b1r, w2, b2r, wah, waf, bar):
    grid = BATCH // BT
    emb_spec = pl.BlockSpec((BT, EMB), lambda i: (i, 0))

    def wspec(shape):
        return pl.BlockSpec(shape, lambda i: (0, 0))

    return pl.pallas_call(
        _tc_body,
        grid=(grid,),
        in_specs=[
            emb_spec, emb_spec, emb_spec, emb_spec,
            wspec((EMB, 32)), wspec((EMB, 32)), wspec((1, 32)),
            wspec((32, EMB)), wspec((1, EMB)),
            wspec((EMB, 1)), wspec((EMB, 1)), wspec((1, 1)),
        ],
        out_specs=pl.BlockSpec((BT, 1), lambda i: (i, 0)),
        out_shape=jax.ShapeDtypeStruct((BATCH, 1), jnp.float32),
    )(u_mlp, i_mlp, u_mf, i_mf, w1u, w1i, b1r, w2, b2r, wah, waf, bar)


def kernel(user_input, item_input, emb_user_mlp, emb_item_mlp,
           emb_user_mf, emb_item_mf, W1, b1, W2, b2, Wa, ba):
    user2d = user_input.astype(jnp.int32).reshape(IDX_ROWS, CHUNK)
    item2d = item_input.astype(jnp.int32).reshape(IDX_ROWS, CHUNK)
    u_mlp, i_mlp, u_mf, i_mf = _sc_gather(
        user2d, item2d, emb_user_mlp, emb_item_mlp, emb_user_mf, emb_item_mf)
    w1u, w1i = W1[:EMB], W1[EMB:]
    wah, waf = Wa[:EMB], Wa[EMB:]
    return _tc_dense(
        u_mlp, i_mlp, u_mf, i_mf,
        w1u, w1i, b1.reshape(1, 32),
        W2, b2.reshape(1, EMB),
        wah, waf, ba.reshape(1, 1))
